# baseline (speedup 1.0000x reference)
import functools

import jax
import jax.numpy as jnp
from jax import lax
from jax.experimental import pallas as pl
from jax.experimental.pallas import tpu as pltpu
from jax.experimental.pallas import tpu_sc as plsc

N = 50000
D = 256
NZ_PAD = 128
S = 1024
BN = 5000
G = N // BN

NW = 32
N_PAD = 50176
CHUNK = N_PAD // NW
ROWS = CHUNK // 16
PIECE = 392


def _mlp_body(emb_ref, w1a_ref, b1_ref, pos_ref, an_ref, batch_ref,
              w1b_ref, w2_ref, b2_ref, w3_ref, b3_ref, e_ref, m_ref):
    i = pl.program_id(0)

    @pl.when(i == 0)
    def _compute_m():
        m_ref[...] = (
            jnp.dot(emb_ref[...], w1a_ref[...],
                    preferred_element_type=jnp.float32)
            + b1_ref[...]
        ).astype(jnp.bfloat16)

    an = an_ref[...]
    onehot_an = (an == jax.lax.broadcasted_iota(jnp.int32, (1, NZ_PAD), 1)
                 ).astype(jnp.bfloat16)
    pre1 = (
        jnp.dot(onehot_an, m_ref[...], preferred_element_type=jnp.float32)
        + jnp.dot(pos_ref[...], w1b_ref[...], preferred_element_type=jnp.float32)
    )
    x1 = (0.5 * pre1) * (1.0 + jnp.tanh(0.5 * pre1))
    pre2 = jnp.dot(x1.astype(jnp.bfloat16), w2_ref[...],
                   preferred_element_type=jnp.float32) + b2_ref[...]
    x2 = (0.5 * pre2) * (1.0 + jnp.tanh(0.5 * pre2))
    e_ref[...] = (jnp.dot(x2.astype(jnp.bfloat16), w3_ref[...],
                          preferred_element_type=jnp.float32)
                  + b3_ref[...])
    _ = batch_ref


def _sc_segsum(e_hbm, ids_hbm, zeros_hbm, out_hbm, e_v, ids_v, shared, sem):
    c = lax.axis_index("c")
    s = lax.axis_index("s")
    wid = s * 2 + c
    base = wid * CHUNK

    @pl.when(s == 0)
    def _zero_shared():
        pltpu.sync_copy(zeros_hbm, shared)

    plsc.subcore_barrier()
    for p in range(CHUNK // PIECE):
        off = base + p * PIECE
        pltpu.async_copy(ids_hbm.at[pl.ds(off, PIECE)], ids_v, sem).wait()
        pltpu.async_copy(e_hbm.at[pl.ds(off, PIECE)], e_v, sem).wait()
        pltpu.async_copy(e_v, shared.at[ids_v], sem, add=True).wait()
    plsc.subcore_barrier()

    @pl.when(s == 0)
    def _writeback():
        pltpu.sync_copy(shared, out_hbm.at[c])


def _epilogue_body(partials_ref, out_ref):
    x = partials_ref[...]
    out_ref[...] = jnp.sum(jnp.sum(x, axis=2), axis=0, keepdims=True)


@functools.partial(jax.jit, static_argnames=())
def kernel(pos, emb, W1, b1, W2, b2, W3, b3, atomic_numbers, batch):
    pos_pad = jnp.pad(pos.astype(jnp.bfloat16), ((0, 0), (0, 5)))
    emb_pad = jnp.pad(emb, ((0, NZ_PAD - emb.shape[0]), (0, 0)))
    W1a = W1[:D, :]
    W1b = jnp.pad(W1[D:, :].astype(jnp.bfloat16), ((0, 5), (0, 0)))
    an2d = atomic_numbers.astype(jnp.int32).reshape(N, 1)
    batch2d = batch.astype(jnp.int32).reshape(N, 1)

    e2d = pl.pallas_call(
        _mlp_body,
        grid=(G,),
        in_specs=[
            pl.BlockSpec((NZ_PAD, D), lambda i: (0, 0)),
            pl.BlockSpec((D, D), lambda i: (0, 0)),
            pl.BlockSpec((1, D), lambda i: (0, 0)),
            pl.BlockSpec((BN, 8), lambda i: (i, 0)),
            pl.BlockSpec((BN, 1), lambda i: (i, 0)),
            pl.BlockSpec((BN, 1), lambda i: (i, 0)),
            pl.BlockSpec((8, D), lambda i: (0, 0)),
            pl.BlockSpec((D, D), lambda i: (0, 0)),
            pl.BlockSpec((1, D), lambda i: (0, 0)),
            pl.BlockSpec((D, 1), lambda i: (0, 0)),
            pl.BlockSpec((1, 1), lambda i: (0, 0)),
        ],
        out_specs=pl.BlockSpec((BN, 1), lambda i: (i, 0)),
        out_shape=jax.ShapeDtypeStruct((N, 1), jnp.float32),
        scratch_shapes=[pltpu.VMEM((NZ_PAD, D), jnp.bfloat16)],
    )(emb_pad, W1a, b1.reshape(1, D), pos_pad, an2d, batch2d, W1b,
      W2.astype(jnp.bfloat16), b2.reshape(1, D),
      W3.astype(jnp.bfloat16), b3.reshape(1, 1))

    e_rows = jnp.pad(e2d, ((0, N_PAD - N), (0, 7)))
    ids_flat = jnp.pad(batch2d.reshape(N), (0, N_PAD - N))
    zrows = jnp.zeros((S, 8), jnp.float32)

    sc_call = functools.partial(
        pl.kernel,
        mesh=plsc.VectorSubcoreMesh(core_axis_name="c", subcore_axis_name="s"),
        compiler_params=pltpu.CompilerParams(use_tc_tiling_on_sc=False),
        out_type=jax.ShapeDtypeStruct((2, S, 8), jnp.float32),
        scratch_types=[
            pltpu.VMEM((PIECE, 8), jnp.float32),
            pltpu.VMEM((PIECE,), jnp.int32),
            pltpu.VMEM_SHARED((S, 8), jnp.float32),
            pltpu.SemaphoreType.DMA,
        ],
    )(_sc_segsum)
    partials = sc_call(e_rows, ids_flat, zrows)

    out = pl.pallas_call(
        _epilogue_body,
        out_shape=jax.ShapeDtypeStruct((1, S), jnp.float32),
    )(partials)

    return out.reshape(S)


# speedup vs baseline: 1.0207x; 1.0207x over previous
import functools

import jax
import jax.numpy as jnp
from jax import lax
from jax.experimental import pallas as pl
from jax.experimental.pallas import tpu as pltpu
from jax.experimental.pallas import tpu_sc as plsc

N = 50000
D = 256
NZ_PAD = 128
S = 1024
BN = 5000
G = N // BN

NW = 32
N_PAD = 50176
CHUNK = N_PAD // NW
ROWS = CHUNK // 16
PIECE = 392


def _mlp_body(emb_ref, w1a_ref, b1_ref, pos_ref, an_ref, batch_ref,
              w1b_ref, w2_ref, b2_ref, w3_ref, b3_ref, e_ref, m_ref):
    i = pl.program_id(0)

    @pl.when(i == 0)
    def _compute_m():
        m_ref[...] = (
            jnp.dot(emb_ref[...], w1a_ref[...],
                    preferred_element_type=jnp.float32)
            + b1_ref[...]
        ).astype(jnp.bfloat16)

    an = an_ref[...]
    onehot_an = (an == jax.lax.broadcasted_iota(jnp.int32, (1, NZ_PAD), 1)
                 ).astype(jnp.bfloat16)
    pre1 = (
        jnp.dot(onehot_an, m_ref[...], preferred_element_type=jnp.float32)
        + jnp.dot(pos_ref[...], w1b_ref[...], preferred_element_type=jnp.float32)
    )
    x1 = (0.5 * pre1) * (1.0 + jnp.tanh(0.5 * pre1))
    pre2 = jnp.dot(x1.astype(jnp.bfloat16), w2_ref[...],
                   preferred_element_type=jnp.float32) + b2_ref[...]
    x2 = (0.5 * pre2) * (1.0 + jnp.tanh(0.5 * pre2))
    e_ref[...] = (jnp.dot(x2.astype(jnp.bfloat16), w3_ref[...],
                          preferred_element_type=jnp.float32)
                  + b3_ref[...])
    _ = batch_ref


def _sc_segsum(e_hbm, ids_hbm, zeros_hbm, out_hbm, e_v, ids_v, shared, sem,
               sem2):
    c = lax.axis_index("c")
    s = lax.axis_index("s")
    wid = s * 2 + c
    base = wid * CHUNK

    @pl.when(s == 0)
    def _zero_shared():
        pltpu.sync_copy(zeros_hbm, shared)

    plsc.subcore_barrier()
    c_ids = pltpu.async_copy(ids_hbm.at[pl.ds(base, CHUNK)], ids_v, sem)
    c_e = pltpu.async_copy(e_hbm.at[pl.ds(base, CHUNK)], e_v, sem2)
    c_ids.wait()
    c_e.wait()
    pltpu.async_copy(e_v, shared.at[ids_v], sem, add=True).wait()
    plsc.subcore_barrier()

    @pl.when(s == 0)
    def _writeback():
        pltpu.sync_copy(shared, out_hbm.at[c])


def _epilogue_body(partials_ref, out_ref):
    x = partials_ref[...]
    out_ref[...] = jnp.sum(jnp.sum(x, axis=2), axis=0, keepdims=True)


@functools.partial(jax.jit, static_argnames=())
def kernel(pos, emb, W1, b1, W2, b2, W3, b3, atomic_numbers, batch):
    pos_pad = jnp.pad(pos.astype(jnp.bfloat16), ((0, 0), (0, 5)))
    emb_pad = jnp.pad(emb, ((0, NZ_PAD - emb.shape[0]), (0, 0)))
    W1a = W1[:D, :]
    W1b = jnp.pad(W1[D:, :].astype(jnp.bfloat16), ((0, 5), (0, 0)))
    an2d = atomic_numbers.astype(jnp.int32).reshape(N, 1)
    batch2d = batch.astype(jnp.int32).reshape(N, 1)

    e2d = pl.pallas_call(
        _mlp_body,
        grid=(G,),
        in_specs=[
            pl.BlockSpec((NZ_PAD, D), lambda i: (0, 0)),
            pl.BlockSpec((D, D), lambda i: (0, 0)),
            pl.BlockSpec((1, D), lambda i: (0, 0)),
            pl.BlockSpec((BN, 8), lambda i: (i, 0)),
            pl.BlockSpec((BN, 1), lambda i: (i, 0)),
            pl.BlockSpec((BN, 1), lambda i: (i, 0)),
            pl.BlockSpec((8, D), lambda i: (0, 0)),
            pl.BlockSpec((D, D), lambda i: (0, 0)),
            pl.BlockSpec((1, D), lambda i: (0, 0)),
            pl.BlockSpec((D, 1), lambda i: (0, 0)),
            pl.BlockSpec((1, 1), lambda i: (0, 0)),
        ],
        out_specs=pl.BlockSpec((BN, 1), lambda i: (i, 0)),
        out_shape=jax.ShapeDtypeStruct((N, 1), jnp.float32),
        scratch_shapes=[pltpu.VMEM((NZ_PAD, D), jnp.bfloat16)],
    )(emb_pad, W1a, b1.reshape(1, D), pos_pad, an2d, batch2d, W1b,
      W2.astype(jnp.bfloat16), b2.reshape(1, D),
      W3.astype(jnp.bfloat16), b3.reshape(1, 1))

    e_rows = jnp.pad(e2d, ((0, N_PAD - N), (0, 7)))
    ids_flat = jnp.pad(batch2d.reshape(N), (0, N_PAD - N))
    zrows = jnp.zeros((S, 8), jnp.float32)

    sc_call = functools.partial(
        pl.kernel,
        mesh=plsc.VectorSubcoreMesh(core_axis_name="c", subcore_axis_name="s"),
        compiler_params=pltpu.CompilerParams(use_tc_tiling_on_sc=False),
        out_type=jax.ShapeDtypeStruct((2, S, 8), jnp.float32),
        scratch_types=[
            pltpu.VMEM((CHUNK, 8), jnp.float32),
            pltpu.VMEM((CHUNK,), jnp.int32),
            pltpu.VMEM_SHARED((S, 8), jnp.float32),
            pltpu.SemaphoreType.DMA,
            pltpu.SemaphoreType.DMA,
        ],
    )(_sc_segsum)
    partials = sc_call(e_rows, ids_flat, zrows)

    out = pl.pallas_call(
        _epilogue_body,
        out_shape=jax.ShapeDtypeStruct((1, S), jnp.float32),
    )(partials)

    return out.reshape(S)
